# baseline (speedup 1.0000x reference)
import functools

import jax
import jax.numpy as jnp
from jax import lax
from jax.experimental import pallas as pl
from jax.experimental.pallas import tpu as pltpu
from jax.experimental.pallas import tpu_sc as plsc

CHUNK = 128
NW = 32
RB = 1024


def _pack(v):
    return jnp.concatenate(
        [v[128 * a:128 * (a + 1), :] for a in range(8)], axis=1)


def _pre_body(x_ref, w_ref, b_ref, c_ref, wc_ref, bc_ref,
              self_ref, hxx_ref, cnew_ref):
    xb = x_ref[...]
    hid = w_ref.shape[1] // 2
    h2 = jnp.maximum(
        jnp.dot(xb, w_ref[...], preferred_element_type=jnp.float32)
        + b_ref[...], 0.0)
    self_ref[...] = _pack(h2[:, :hid])
    hxx_ref[...] = _pack(h2[:, hid:])
    cnew_ref[...] = jnp.maximum(
        jnp.dot(c_ref[...], wc_ref[...], preferred_element_type=jnp.float32)
        + bc_ref[...], 0.0)


def _tc_pre(x, W_x, b_x, W_xx, b_xx, c, W_c, b_c, n2):
    n, in_f = x.shape
    hid = W_x.shape[1]
    ctx = c.shape[1]
    w2 = jnp.concatenate([W_x, W_xx], axis=1)
    b2 = jnp.concatenate([b_x, b_xx]).reshape(1, 2 * hid)
    grid = (n2 // RB,)
    return pl.pallas_call(
        _pre_body,
        grid=grid,
        in_specs=[
            pl.BlockSpec((RB, in_f), lambda i: (i, 0)),
            pl.BlockSpec((in_f, 2 * hid), lambda i: (0, 0)),
            pl.BlockSpec((1, 2 * hid), lambda i: (0, 0)),
            pl.BlockSpec((1, ctx), lambda i: (0, 0)),
            pl.BlockSpec((ctx, hid), lambda i: (0, 0)),
            pl.BlockSpec((1, hid), lambda i: (0, 0)),
        ],
        out_specs=[
            pl.BlockSpec((RB // 8, 128), lambda i: (i, 0)),
            pl.BlockSpec((RB // 8, 128), lambda i: (i, 0)),
            pl.BlockSpec((1, hid), lambda i: (0, 0)),
        ],
        out_shape=[
            jax.ShapeDtypeStruct((n2 // 8, 128), jnp.float32),
            jax.ShapeDtypeStruct((n2 // 8, 128), jnp.float32),
            jax.ShapeDtypeStruct((1, hid), jnp.float32),
        ],
    )(x, w2, b2, c, W_c, b_c.reshape(1, -1))


def _sc_aggregate(hxx, ei3, n_pad, base_cpt, n_extra):
    hid = hxx.shape[1]
    rps = n_pad // 16
    assert base_cpt % 2 == 0
    mesh = plsc.VectorSubcoreMesh(core_axis_name="c", subcore_axis_name="s")

    @functools.partial(
        pl.kernel,
        mesh=mesh,
        out_type=jax.ShapeDtypeStruct((2 * n_pad, hid), jnp.float32),
        scratch_types=[
            pltpu.VMEM((base_cpt + 4, 2, CHUNK), jnp.int32),
            [pltpu.VMEM((CHUNK, hid), jnp.float32) for _ in range(2)],
            pltpu.VMEM((rps, hid), jnp.float32),
            pltpu.VMEM_SHARED((n_pad, hid), jnp.float32),
            pltpu.SemaphoreType.DMA,
        ],
        compiler_params=pltpu.CompilerParams(use_tc_tiling_on_sc=False),
    )
    def k(hxx_hbm, ei_hbm, out_hbm, idx_v, bufs, zero_v, acc_sh, sem):
        cid = lax.axis_index("c")
        sid = lax.axis_index("s")
        wid = cid * 16 + sid

        def zbody(i, carry):
            zero_v[i, :] = jnp.zeros((hid,), jnp.float32)
            return carry
        lax.fori_loop(0, rps, zbody, 0)
        pltpu.sync_copy(zero_v, acc_sh.at[pl.ds(sid * rps, rps)])

        for r in range(base_cpt, base_cpt + 4):
            for kk in range(2):
                for m in range(8):
                    idx_v[r, kk, pl.ds(16 * m, 16)] = jnp.zeros(
                        (16,), jnp.int32)
        pltpu.sync_copy(ei_hbm.at[pl.ds(wid * base_cpt, base_cpt)],
                        idx_v.at[pl.ds(0, base_cpt)])
        if n_extra:
            @pl.when(wid < n_extra)
            def _():
                pltpu.sync_copy(ei_hbm.at[NW * base_cpt + wid],
                                idx_v.at[base_cpt])
        plsc.subcore_barrier()

        def xform(j):
            for kk in range(2):
                for m in range(8):
                    v = idx_v[j, kk, pl.ds(16 * m, 16)]
                    idx_v[j, kk, pl.ds(16 * m, 16)] = (
                        (v & -1024) | ((v & 127) << 3) | ((v >> 7) & 7))

        def fire(j, b):
            pltpu.async_copy(hxx_hbm.at[idx_v.at[j, 0]], bufs[b], sem)

        def drain(b):
            pltpu.make_async_copy(
                hxx_hbm.at[pl.ds(0, CHUNK)], bufs[b], sem).wait()

        def scat(j, b):
            pltpu.sync_copy(bufs[b], acc_sh.at[idx_v.at[j, 1]], add=True)

        xform(0)
        xform(1)
        fire(0, 0)
        fire(1, 1)

        def body(i, carry):
            a = i * 2
            xform(a + 2)
            xform(a + 3)
            drain(0)
            scat(a, 0)
            fire(a + 2, 0)
            drain(1)
            scat(a + 1, 1)
            fire(a + 3, 1)
            return carry

        lax.fori_loop(0, base_cpt // 2, body, 0)
        drain(0)
        if n_extra:
            @pl.when(wid < n_extra)
            def _():
                scat(base_cpt, 0)
        drain(1)
        plsc.subcore_barrier()
        pltpu.sync_copy(acc_sh.at[pl.ds(sid * rps, rps)],
                        out_hbm.at[pl.ds(cid * n_pad + sid * rps, rps)])

    return k(hxx, ei3)


def _post_body(p0_ref, p1_ref, self_ref, wp_ref, bp_ref, out_ref):
    s = p0_ref[...] + p1_ref[...] + self_ref[...]
    wp = wp_ref[...]
    out_ref[...] = jnp.concatenate(
        [jnp.dot(s[:, 16 * a:16 * (a + 1)], wp,
                 preferred_element_type=jnp.float32) for a in range(8)],
        axis=0) + bp_ref[...]


def _tc_post(partials_p, self_p, W_pool, b_pool, n, n_pad):
    hid, out_f = W_pool.shape
    nblk = n_pad // RB
    grid = (nblk,)
    return pl.pallas_call(
        _post_body,
        grid=grid,
        in_specs=[
            pl.BlockSpec((RB // 8, 128), lambda i: (i, 0)),
            pl.BlockSpec((RB // 8, 128), lambda i, nb=nblk: (i + nb, 0)),
            pl.BlockSpec((RB // 8, 128), lambda i: (i, 0)),
            pl.BlockSpec((hid, out_f), lambda i: (0, 0)),
            pl.BlockSpec((1, out_f), lambda i: (0, 0)),
        ],
        out_specs=pl.BlockSpec((RB, out_f), lambda i: (i, 0)),
        out_shape=jax.ShapeDtypeStruct((n, out_f), jnp.float32),
    )(partials_p, partials_p, self_p, W_pool, b_pool.reshape(1, -1))


def kernel(x, c, edge_index, W_x, b_x, W_xx, b_xx, W_c, b_c, W_pool, b_pool):
    n = x.shape[0]
    hid = W_x.shape[1]
    e = edge_index.shape[1]
    assert e % CHUNK == 0
    n_chunks = e // CHUNK
    base_cpt = n_chunks // NW
    n_extra = n_chunks - base_cpt * NW

    n_pad = -(-n // RB) * RB

    self_p, hxx_p, c_new = _tc_pre(x, W_x, b_x, W_xx, b_xx, c, W_c, b_c,
                                   n_pad)

    ei3 = jnp.transpose(
        edge_index.astype(jnp.int32).reshape(2, n_chunks, CHUNK), (1, 0, 2))
    partials = _sc_aggregate(hxx_p.reshape(n_pad, hid), ei3, n_pad,
                             base_cpt, n_extra)
    partials_p = partials.reshape(2 * n_pad // 8, 128)
    x_new = _tc_post(partials_p, self_p, W_pool, b_pool, n, n_pad)
    return (x_new, c_new)


# speedup vs baseline: 1.0357x; 1.0357x over previous
import functools

import jax
import jax.numpy as jnp
from jax import lax
from jax.experimental import pallas as pl
from jax.experimental.pallas import tpu as pltpu
from jax.experimental.pallas import tpu_sc as plsc

CHUNK = 128
NW = 32
RB = 1024


def _pack(v):
    return jnp.concatenate(
        [v[128 * a:128 * (a + 1), :] for a in range(8)], axis=1)


def _pre_body(x_ref, w_ref, b_ref, c_ref, wc_ref, bc_ref,
              self_ref, hxx_ref, cnew_ref):
    xb = x_ref[...]
    hid = w_ref.shape[1] // 2
    h2 = jnp.maximum(
        jnp.dot(xb, w_ref[...], preferred_element_type=jnp.float32)
        + b_ref[...], 0.0)
    self_ref[...] = _pack(h2[:, :hid])
    hxx_ref[...] = _pack(h2[:, hid:])
    cnew_ref[...] = jnp.maximum(
        jnp.dot(c_ref[...], wc_ref[...], preferred_element_type=jnp.float32)
        + bc_ref[...], 0.0)


def _tc_pre(x, W_x, b_x, W_xx, b_xx, c, W_c, b_c, n2):
    n, in_f = x.shape
    hid = W_x.shape[1]
    ctx = c.shape[1]
    w2 = jnp.concatenate([W_x, W_xx], axis=1)
    b2 = jnp.concatenate([b_x, b_xx]).reshape(1, 2 * hid)
    grid = (n2 // RB,)
    return pl.pallas_call(
        _pre_body,
        grid=grid,
        in_specs=[
            pl.BlockSpec((RB, in_f), lambda i: (i, 0)),
            pl.BlockSpec((in_f, 2 * hid), lambda i: (0, 0)),
            pl.BlockSpec((1, 2 * hid), lambda i: (0, 0)),
            pl.BlockSpec((1, ctx), lambda i: (0, 0)),
            pl.BlockSpec((ctx, hid), lambda i: (0, 0)),
            pl.BlockSpec((1, hid), lambda i: (0, 0)),
        ],
        out_specs=[
            pl.BlockSpec((RB // 8, 128), lambda i: (i, 0)),
            pl.BlockSpec((RB // 8, 128), lambda i: (i, 0)),
            pl.BlockSpec((1, hid), lambda i: (0, 0)),
        ],
        out_shape=[
            jax.ShapeDtypeStruct((n2 // 8, 128), jnp.float32),
            jax.ShapeDtypeStruct((n2 // 8, 128), jnp.float32),
            jax.ShapeDtypeStruct((1, hid), jnp.float32),
        ],
    )(x, w2, b2, c, W_c, b_c.reshape(1, -1))


def _sc_aggregate(hxx, ei3, n_pad, base_cpt, n_extra):
    hid = hxx.shape[1]
    rps = n_pad // 16
    cpt_max = base_cpt + (1 if n_extra else 0)
    mesh = plsc.VectorSubcoreMesh(core_axis_name="c", subcore_axis_name="s")

    @functools.partial(
        pl.kernel,
        mesh=mesh,
        out_type=jax.ShapeDtypeStruct((2 * n_pad, hid), jnp.float32),
        scratch_types=[
            pltpu.VMEM((cpt_max + 1, 2, CHUNK), jnp.int32),
            pltpu.VMEM((CHUNK, hid), jnp.float32),
            pltpu.VMEM((rps, hid), jnp.float32),
            pltpu.VMEM_SHARED((n_pad, hid), jnp.float32),
            pltpu.SemaphoreType.DMA,
        ],
        compiler_params=pltpu.CompilerParams(use_tc_tiling_on_sc=False),
    )
    def k(hxx_hbm, ei_hbm, out_hbm, idx_v, rows_v, zero_v, acc_sh, sem):
        cid = lax.axis_index("c")
        sid = lax.axis_index("s")
        wid = cid * 16 + sid

        def zbody(i, carry):
            zero_v[i, :] = jnp.zeros((hid,), jnp.float32)
            return carry
        lax.fori_loop(0, rps, zbody, 0)
        pltpu.sync_copy(zero_v, acc_sh.at[pl.ds(sid * rps, rps)])

        pltpu.sync_copy(ei_hbm.at[pl.ds(wid * base_cpt, base_cpt)],
                        idx_v.at[pl.ds(0, base_cpt)])
        n_ch = base_cpt
        if n_extra:
            @pl.when(wid < n_extra)
            def _():
                pltpu.sync_copy(ei_hbm.at[NW * base_cpt + wid],
                                idx_v.at[base_cpt])
            n_ch = base_cpt + (wid < n_extra).astype(jnp.int32)
        plsc.subcore_barrier()

        def xform(j):
            for kk in range(2):
                for m in range(8):
                    v = idx_v[j, kk, pl.ds(16 * m, 16)]
                    idx_v[j, kk, pl.ds(16 * m, 16)] = (
                        (v & -1024) | ((v & 127) << 3) | ((v >> 7) & 7))

        xform(0)

        def body(j, carry):
            cp = pltpu.async_copy(hxx_hbm.at[idx_v.at[j, 0]], rows_v, sem)
            xform(j + 1)
            cp.wait()
            pltpu.sync_copy(rows_v, acc_sh.at[idx_v.at[j, 1]], add=True)
            return carry

        lax.fori_loop(0, n_ch, body, 0)
        plsc.subcore_barrier()
        pltpu.sync_copy(acc_sh.at[pl.ds(sid * rps, rps)],
                        out_hbm.at[pl.ds(cid * n_pad + sid * rps, rps)])

    return k(hxx, ei3)


def _post_body(p0_ref, p1_ref, self_ref, wp_ref, bp_ref, out_ref):
    s = p0_ref[...] + p1_ref[...] + self_ref[...]
    wp = wp_ref[...]
    out_ref[...] = jnp.concatenate(
        [jnp.dot(s[:, 16 * a:16 * (a + 1)], wp,
                 preferred_element_type=jnp.float32) for a in range(8)],
        axis=0) + bp_ref[...]


def _tc_post(partials_p, self_p, W_pool, b_pool, n, n_pad):
    hid, out_f = W_pool.shape
    nblk = n_pad // RB
    grid = (nblk,)
    return pl.pallas_call(
        _post_body,
        grid=grid,
        in_specs=[
            pl.BlockSpec((RB // 8, 128), lambda i: (i, 0)),
            pl.BlockSpec((RB // 8, 128), lambda i, nb=nblk: (i + nb, 0)),
            pl.BlockSpec((RB // 8, 128), lambda i: (i, 0)),
            pl.BlockSpec((hid, out_f), lambda i: (0, 0)),
            pl.BlockSpec((1, out_f), lambda i: (0, 0)),
        ],
        out_specs=pl.BlockSpec((RB, out_f), lambda i: (i, 0)),
        out_shape=jax.ShapeDtypeStruct((n, out_f), jnp.float32),
    )(partials_p, partials_p, self_p, W_pool, b_pool.reshape(1, -1))


def kernel(x, c, edge_index, W_x, b_x, W_xx, b_xx, W_c, b_c, W_pool, b_pool):
    n = x.shape[0]
    hid = W_x.shape[1]
    e = edge_index.shape[1]
    assert e % CHUNK == 0
    n_chunks = e // CHUNK
    base_cpt = n_chunks // NW
    n_extra = n_chunks - base_cpt * NW

    n_pad = -(-n // RB) * RB

    self_p, hxx_p, c_new = _tc_pre(x, W_x, b_x, W_xx, b_xx, c, W_c, b_c,
                                   n_pad)

    ei3 = jnp.transpose(
        edge_index.astype(jnp.int32).reshape(2, n_chunks, CHUNK), (1, 0, 2))
    partials = _sc_aggregate(hxx_p.reshape(n_pad, hid), ei3, n_pad,
                             base_cpt, n_extra)
    partials_p = partials.reshape(2 * n_pad // 8, 128)
    x_new = _tc_post(partials_p, self_p, W_pool, b_pool, n, n_pad)
    return (x_new, c_new)
